# baseline (speedup 1.0000x reference)
import functools

import jax
import jax.numpy as jnp
from jax import lax
from jax.experimental import pallas as pl
from jax.experimental.pallas import tpu as pltpu
from jax.experimental.pallas import tpu_sc as plsc

NUM_CLASSES = 4
MARGIN = 1.0

B = 2
C = 32
N = 64 * 64 * 64
K = NUM_CLASSES
L = 16
NC = 2
NS = 16
NW = NC * NS
CHUNK = N // NW
NVEC = CHUNK // L
SLOT = K * L
NSLOT = C + 1
PB = NSLOT * SLOT
PTOT = B * PB


GC = 4
NG = B * C // GC
GPB = C // GC
R = 4


def _sc_body(feat_hbm, tgt_hbm, out_hbm, idx_v, fbuf, acc_v, sem0, sem1):
    wid = lax.axis_index("s") * NC + lax.axis_index("c")
    base = wid * CHUNK
    sems = (sem0, sem1)

    tcp0 = pltpu.async_copy(
        tgt_hbm.at[0, pl.ds(base, CHUNK)], idx_v.at[pl.ds(0, CHUNK)], sem0
    )
    tcp1 = pltpu.async_copy(
        tgt_hbm.at[1, pl.ds(base, CHUNK)], idx_v.at[pl.ds(CHUNK, CHUNK)], sem1
    )

    zeros = jnp.zeros((L,), jnp.float32)

    @plsc.parallel_loop(0, R * PTOT // L, unroll=4)
    def _zero(i):
        acc_v[pl.ds(i * L, L)] = zeros

    tcp0.wait()
    tcp1.wait()

    lane = lax.iota(jnp.int32, L)
    ones = jnp.ones((L,), jnp.float32)

    for bb in range(B):

        @plsc.parallel_loop(0, NVEC, unroll=4)
        def _prep(i, bb=bb):
            sl = pl.ds(bb * CHUNK + i * L, L)
            idx = idx_v[sl] * L + lane + (bb * PB)
            idx_v[sl] = idx
            roff = (i & (R - 1)) * PTOT
            plsc.addupdate_scatter(acc_v, [idx + (roff + C * SLOT)], ones)

    def start(g):
        bb = g // GPB
        c0 = (g % GPB) * GC
        return [
            pltpu.async_copy(
                feat_hbm.at[bb, c0 + cs, pl.ds(base, CHUNK)],
                fbuf.at[g % 2, cs],
                sems[g % 2],
            )
            for cs in range(GC)
        ]

    cur = start(0)
    for g in range(NG):
        nxt = start(g + 1) if g + 1 < NG else None
        for cp in cur:
            cp.wait()
        bb = g // GPB
        c0 = (g % GPB) * GC
        buf = g % 2
        boff = bb * CHUNK

        @plsc.parallel_loop(0, NVEC, unroll=4)
        def _scatter(i, buf=buf, boff=boff, c0=c0):
            voff = i * L
            iv = idx_v[pl.ds(boff + voff, L)]
            ivr = iv + (i & (R - 1)) * PTOT
            for cs in range(GC):
                f = fbuf[buf, cs, pl.ds(voff, L)]
                plsc.addupdate_scatter(acc_v, [ivr + (c0 + cs) * SLOT], f)

        cur = nxt

    pltpu.sync_copy(acc_v, out_hbm.at[wid])


_sc_call = functools.partial(
    pl.kernel,
    mesh=plsc.VectorSubcoreMesh(core_axis_name="c", subcore_axis_name="s"),
    out_type=jax.ShapeDtypeStruct((NW, R * PTOT), jnp.float32),
    scratch_types=[
        pltpu.VMEM((B * CHUNK,), jnp.int32),
        pltpu.VMEM((2, GC, CHUNK), jnp.float32),
        pltpu.VMEM((R * PTOT,), jnp.float32),
        pltpu.SemaphoreType.DMA,
        pltpu.SemaphoreType.DMA,
    ],
    compiler_params=pltpu.CompilerParams(needs_layout_passes=False),
)(_sc_body)


def _fin_body(p_ref, loss_ref):
    p = p_ref[...]
    s = jnp.sum(p, axis=(0, 4))
    sums = s[:, :C, :]
    counts = s[:, C, :]
    centers = sums / jnp.maximum(counts, 1.0)[:, None, :]
    valid = counts > 0
    total = jnp.float32(0.0)
    pairs = jnp.float32(0.0)
    for i in range(NUM_CLASSES):
        for j in range(i + 1, NUM_CLASSES):
            diff = centers[:, :, i] - centers[:, :, j]
            dist = jnp.sqrt(jnp.sum(diff * diff, axis=1))
            hinge = jnp.maximum(MARGIN - dist, 0.0)
            m = jnp.logical_and(valid[:, i], valid[:, j]).astype(jnp.float32)
            total = total + jnp.sum(hinge * m)
            pairs = pairs + jnp.sum(m)
    val = jnp.where(pairs > 0, total / jnp.maximum(pairs, 1.0), 0.0)
    loss_ref[...] = val.reshape(1, 1)


def _finalize(q):
    return pl.pallas_call(
        _fin_body,
        out_shape=jax.ShapeDtypeStruct((1, 1), jnp.float32),
    )(q)


def kernel(features, predictions, targets):
    del predictions
    feats = features.reshape(B, C, N)
    tgt = targets.reshape(B, N)
    partial = _sc_call(feats, tgt)
    q = partial.reshape(NW * R, B, NSLOT, K, L)
    loss = _finalize(q)
    return loss[0, 0]


# speedup vs baseline: 1.2650x; 1.2650x over previous
import functools

import jax
import jax.numpy as jnp
from jax import lax
from jax.experimental import pallas as pl
from jax.experimental.pallas import tpu as pltpu
from jax.experimental.pallas import tpu_sc as plsc

NUM_CLASSES = 4
MARGIN = 1.0

B = 2
C = 32
N = 64 * 64 * 64
K = NUM_CLASSES

CT = 16
CS = C - CT

L = 16
NC = 2
NS = 16
NW = NC * NS
CHUNK = N // NW
NVEC = CHUNK // L
SLOT = K * L
NSLOT = CS + 1
PB = NSLOT * SLOT
PTOT = B * PB
GC = 4
GPB = CS // GC
NG = B * GPB
R = 4


def _sc_body(feat_hbm, tgt_hbm, out_hbm, idx_v, fbuf, acc_v, sem0, sem1):
    wid = lax.axis_index("s") * NC + lax.axis_index("c")
    base = wid * CHUNK
    sems = (sem0, sem1)

    tcp0 = pltpu.async_copy(
        tgt_hbm.at[0, pl.ds(base, CHUNK)], idx_v.at[pl.ds(0, CHUNK)], sem0
    )
    tcp1 = pltpu.async_copy(
        tgt_hbm.at[1, pl.ds(base, CHUNK)], idx_v.at[pl.ds(CHUNK, CHUNK)], sem1
    )

    zeros = jnp.zeros((L,), jnp.float32)

    @plsc.parallel_loop(0, R * PTOT // L, unroll=4)
    def _zero(i):
        acc_v[pl.ds(i * L, L)] = zeros

    tcp0.wait()
    tcp1.wait()

    lane = lax.iota(jnp.int32, L)
    ones = jnp.ones((L,), jnp.float32)

    for bb in range(B):

        @plsc.parallel_loop(0, NVEC, unroll=4)
        def _prep(i, bb=bb):
            sl = pl.ds(bb * CHUNK + i * L, L)
            idx = idx_v[sl] * L + lane + (bb * PB)
            idx_v[sl] = idx
            roff = (i & (R - 1)) * PTOT
            plsc.addupdate_scatter(acc_v, [idx + (roff + CS * SLOT)], ones)

    def start(g):
        bb = g // GPB
        c0 = CT + (g % GPB) * GC
        return pltpu.async_copy(
            feat_hbm.at[bb, pl.ds(c0, GC), pl.ds(base, CHUNK)],
            fbuf.at[g % 2],
            sems[g % 2],
        )

    cur = start(0)
    for g in range(NG):
        nxt = start(g + 1) if g + 1 < NG else None
        cur.wait()
        buf = g % 2
        boff = (g // GPB) * CHUNK
        s0 = (g % GPB) * GC * SLOT

        @plsc.parallel_loop(0, NVEC, unroll=4)
        def _scatter(i, buf=buf, boff=boff, s0=s0):
            voff = i * L
            iv = idx_v[pl.ds(boff + voff, L)]
            ivr = iv + (i & (R - 1)) * PTOT
            for cs in range(GC):
                f = fbuf[buf, cs, pl.ds(voff, L)]
                plsc.addupdate_scatter(acc_v, [ivr + (s0 + cs * SLOT)], f)

        cur = nxt

    pltpu.sync_copy(acc_v, out_hbm.at[wid])


_sc_call = functools.partial(
    pl.kernel,
    mesh=plsc.VectorSubcoreMesh(core_axis_name="c", subcore_axis_name="s"),
    out_type=jax.ShapeDtypeStruct((NW, R * PTOT), jnp.float32),
    scratch_types=[
        pltpu.VMEM((B * CHUNK,), jnp.int32),
        pltpu.VMEM((2, GC, CHUNK), jnp.float32),
        pltpu.VMEM((R * PTOT,), jnp.float32),
        pltpu.SemaphoreType.DMA,
        pltpu.SemaphoreType.DMA,
    ],
    compiler_params=pltpu.CompilerParams(needs_layout_passes=False),
)(_sc_body)


NBLK = 8
Nb = N // NBLK


def _tc_body(f_ref, t_ref, sums_ref):
    b = pl.program_id(0)
    n = pl.program_id(1)

    @pl.when(jnp.logical_and(b == 0, n == 0))
    def _init():
        sums_ref[...] = jnp.zeros_like(sums_ref)

    f = f_ref[0]
    t = t_ref[0]
    s_total = jnp.sum(f, axis=1, keepdims=True)
    zero = jnp.zeros_like(f)
    s_rest = jnp.zeros_like(s_total)
    for k in range(1, NUM_CLASSES):
        m = t == k
        s_k = jnp.sum(jnp.where(m, f, zero), axis=1, keepdims=True)
        sums_ref[b, :, k : k + 1] += s_k
        s_rest = s_rest + s_k
    sums_ref[b, :, 0:1] += s_total - s_rest


def _tc_call(feats, tgt):
    return pl.pallas_call(
        _tc_body,
        grid=(B, NBLK),
        in_specs=[
            pl.BlockSpec((1, CT, Nb), lambda b, n: (b, 0, n)),
            pl.BlockSpec((1, 1, Nb), lambda b, n: (b, 0, n)),
        ],
        out_specs=pl.BlockSpec((B, CT, K), lambda b, n: (0, 0, 0)),
        out_shape=jax.ShapeDtypeStruct((B, CT, K), jnp.float32),
    )(feats, tgt)


def _fin_body(p_ref, t_ref, loss_ref):
    p = p_ref[...]
    s = jnp.sum(p, axis=(0, 4))
    counts = s[:, CS, :]
    cmax = jnp.maximum(counts, 1.0)[:, None, :]
    centers_sc = s[:, :CS, :] / cmax
    centers_tc = t_ref[...] / cmax
    valid = counts > 0
    total = jnp.float32(0.0)
    pairs = jnp.float32(0.0)
    for i in range(NUM_CLASSES):
        for j in range(i + 1, NUM_CLASSES):
            dt = centers_tc[:, :, i] - centers_tc[:, :, j]
            ds = centers_sc[:, :, i] - centers_sc[:, :, j]
            d2 = jnp.sum(dt * dt, axis=1) + jnp.sum(ds * ds, axis=1)
            dist = jnp.sqrt(d2)
            hinge = jnp.maximum(MARGIN - dist, 0.0)
            m = jnp.logical_and(valid[:, i], valid[:, j]).astype(jnp.float32)
            total = total + jnp.sum(hinge * m)
            pairs = pairs + jnp.sum(m)
    val = jnp.where(pairs > 0, total / jnp.maximum(pairs, 1.0), 0.0)
    loss_ref[...] = val.reshape(1, 1)


def _finalize(q, tc_sums):
    return pl.pallas_call(
        _fin_body,
        out_shape=jax.ShapeDtypeStruct((1, 1), jnp.float32),
    )(q, tc_sums)


def kernel(features, predictions, targets):
    del predictions
    feats = features.reshape(B, C, N)
    tgt2 = targets.reshape(B, N)
    tgt3 = targets.reshape(B, 1, N)
    sc_part = _sc_call(feats, tgt2)
    tc_sums = _tc_call(feats, tgt3)
    q = sc_part.reshape(NW * R, B, NSLOT, K, L)
    loss = _finalize(q, tc_sums)
    return loss[0, 0]


# speedup vs baseline: 2.8911x; 2.2854x over previous
import jax
import jax.numpy as jnp
from jax import lax
from jax.experimental import pallas as pl
from jax.experimental.pallas import tpu as pltpu

NUM_CLASSES = 4
MARGIN = 1.0

B = 2
C = 32
H = W = D = 64
K = NUM_CLASSES
HB = 8


def _tc_body(f_ref, t_ref, sums_ref, counts_ref, loss_ref):
    b = pl.program_id(0)
    n = pl.program_id(1)

    @pl.when(jnp.logical_and(b == 0, n == 0))
    def _init():
        sums_ref[...] = jnp.zeros_like(sums_ref)
        counts_ref[...] = jnp.zeros_like(counts_ref)

    f = f_ref[0]
    t = t_ref[0]
    s_total = jnp.sum(jnp.sum(f, axis=(2, 3)), axis=1, keepdims=True)
    zero = jnp.zeros_like(f)
    s_rest = jnp.zeros_like(s_total)
    n_rest = jnp.zeros((1, 1), jnp.float32)
    for k in range(1, NUM_CLASSES):
        m = t == k
        s_k = jnp.sum(jnp.sum(jnp.where(m, f, zero), axis=(2, 3)), axis=1, keepdims=True)
        n_k = jnp.sum(
            jnp.sum(m.astype(jnp.float32), axis=(2, 3)), axis=1, keepdims=True
        )
        sums_ref[b, :, k : k + 1] += s_k
        counts_ref[b, :, k : k + 1] += n_k
        s_rest = s_rest + s_k
        n_rest = n_rest + n_k
    sums_ref[b, :, 0:1] += s_total - s_rest
    counts_ref[b, :, 0:1] += jnp.float32(HB * W * D) - n_rest

    @pl.when(jnp.logical_and(b == B - 1, n == pl.num_programs(1) - 1))
    def _finalize():
        sums = sums_ref[...]
        counts = counts_ref[...]
        centers = sums / jnp.maximum(counts, 1.0)
        valid = counts[:, 0, :] > 0
        total = jnp.float32(0.0)
        pairs = jnp.float32(0.0)
        for i in range(NUM_CLASSES):
            for j in range(i + 1, NUM_CLASSES):
                diff = centers[:, :, i] - centers[:, :, j]
                dist = jnp.sqrt(jnp.sum(diff * diff, axis=1))
                hinge = jnp.maximum(MARGIN - dist, 0.0)
                m = jnp.logical_and(valid[:, i], valid[:, j]).astype(jnp.float32)
                total = total + jnp.sum(hinge * m)
                pairs = pairs + jnp.sum(m)
        val = jnp.where(pairs > 0, total / jnp.maximum(pairs, 1.0), 0.0)
        loss_ref[...] = val.reshape(1, 1)


def kernel(features, predictions, targets):
    del predictions
    sums, counts, loss = pl.pallas_call(
        _tc_body,
        grid=(B, H // HB),
        in_specs=[
            pl.BlockSpec((1, C, HB, W, D), lambda b, n: (b, 0, n, 0, 0)),
            pl.BlockSpec((1, 1, HB, W, D), lambda b, n: (b, 0, n, 0, 0)),
        ],
        out_specs=[
            pl.BlockSpec((B, C, K), lambda b, n: (0, 0, 0)),
            pl.BlockSpec((B, 1, K), lambda b, n: (0, 0, 0)),
            pl.BlockSpec((1, 1), lambda b, n: (0, 0)),
        ],
        out_shape=[
            jax.ShapeDtypeStruct((B, C, K), jnp.float32),
            jax.ShapeDtypeStruct((B, 1, K), jnp.float32),
            jax.ShapeDtypeStruct((1, 1), jnp.float32),
        ],
    )(features, targets)
    return loss[0, 0]
